# baseline (speedup 1.0000x reference)
import functools

import jax
import jax.numpy as jnp
from jax import lax
from jax.experimental import pallas as pl
from jax.experimental.pallas import tpu as pltpu
from jax.experimental.pallas import tpu_sc as plsc

N = 10000
E = 320000
DN = 128
DE = 16

NC = 2
NS = 16
NW = NC * NS
EPW = E // NW
CH = 80
NCHUNK = EPW // CH
RPT = 640

_mesh = plsc.VectorSubcoreMesh(core_axis_name="c", subcore_axis_name="s")


def _proj_body(nodes_ref, ws_ref, wr_ref, ps_ref, pr_ref):
    n = nodes_ref[...]
    ps_ref[...] = jnp.dot(n, ws_ref[...], preferred_element_type=jnp.float32)
    pr_ref[...] = jnp.dot(n, wr_ref[...], preferred_element_type=jnp.float32)


def _proj(nodes, ws, wr):
    return pl.pallas_call(
        _proj_body,
        out_shape=(
            jax.ShapeDtypeStruct((N, DN), jnp.float32),
            jax.ShapeDtypeStruct((N, DN), jnp.float32),
        ),
    )(nodes, ws, wr)


NSLOT = 8
_NZC = RPT // CH
_NZC_LAST = (N - (NS - 1) * RPT) // CH


def _ranged(sid, fn):
    @pl.when(sid < NS - 1)
    def _():
        for k in range(_NZC):
            fn(k)

    @pl.when(sid == NS - 1)
    def _():
        for k in range(_NZC_LAST):
            fn(k)


@functools.partial(
    pl.kernel,
    out_type=jax.ShapeDtypeStruct((E, DN), jnp.float32),
    mesh=_mesh,
    scratch_types=(
        [pltpu.VMEM((NCHUNK, CH), jnp.int32)] * 2
        + [pltpu.VMEM((CH, DN), jnp.float32)] * NSLOT
        + [pltpu.SemaphoreType.DMA] * (3 * NSLOT)
    ),
)
def _gather(ps_hbm, pr_hbm, snd_hbm, rcv_hbm, gsum_hbm, *refs):
    sidx, ridx = refs[0], refs[1]
    buf = refs[2:2 + NSLOT]
    sems = refs[2 + NSLOT:]
    ga, gb = sems[0:NSLOT], sems[NSLOT:2 * NSLOT]
    wa = sems[2 * NSLOT:3 * NSLOT]

    wid = lax.axis_index("s") * NC + lax.axis_index("c")
    base0 = wid * EPW

    pltpu.sync_copy(snd_hbm.at[wid], sidx)
    pltpu.sync_copy(rcv_hbm.at[wid], ridx)

    def gstart(j, b):
        pltpu.async_copy(ps_hbm.at[sidx.at[j]], buf[b], ga[b])

    def gwait(b):
        pltpu.make_async_copy(ps_hbm.at[sidx.at[0]], buf[b], ga[b]).wait()

    def astart(j, b):
        pltpu.async_copy(pr_hbm.at[ridx.at[j]], buf[b], gb[b], add=True)

    def await_(b):
        pltpu.make_async_copy(pr_hbm.at[ridx.at[0]], buf[b], gb[b]).wait()

    def wstart(j, b):
        pltpu.async_copy(buf[b], gsum_hbm.at[pl.ds(base0 + j * CH, CH)], wa[b])

    def wdrain(b):
        pltpu.make_async_copy(buf[b], gsum_hbm.at[pl.ds(0, CH)], wa[b]).wait()

    for b in range(NSLOT):
        gstart(b, b)

    def body(i, carry):
        for b in range(NSLOT):
            b1 = (b - 1) % NSLOT
            b2 = (b - 2) % NSLOT

            @pl.when(lax.rem(i, NSLOT) == b)
            def _():
                gwait(b)
                astart(i, b)

                @pl.when(i >= 1)
                def _():
                    await_(b1)
                    wstart(i - 1, b1)

                @pl.when((i >= 2) & (i - 2 + NSLOT < NCHUNK))
                def _():
                    wdrain(b2)
                    gstart(i - 2 + NSLOT, b2)

        return carry

    lax.fori_loop(0, NCHUNK, body, 0)
    bl = (NCHUNK - 1) % NSLOT
    await_(bl)
    wstart(NCHUNK - 1, bl)
    wdrain((NCHUNK - 2) % NSLOT)
    wdrain(bl)


BE = 4000


def _edge_body(g_ref, w0g_ref, b0_ref, w0e_ref, w1_ref, b1_ref, w2_ref, b2_ref,
               e_ref, gsum_ref, out_ref, esum_ref):
    gvec = jnp.dot(g_ref[...], w0g_ref[...],
                   preferred_element_type=jnp.float32) + b0_ref[...]
    h0 = jnp.dot(e_ref[...], w0e_ref[...], preferred_element_type=jnp.float32)
    h0 = jnp.maximum(h0 + gsum_ref[...] + gvec, 0.0)
    h1 = jnp.maximum(
        jnp.dot(h0.astype(jnp.bfloat16), w1_ref[...],
                preferred_element_type=jnp.float32) + b1_ref[...], 0.0)
    out = jnp.dot(h1.astype(jnp.bfloat16), w2_ref[...],
                  preferred_element_type=jnp.float32) + b2_ref[...]
    out_ref[...] = out

    @pl.when(pl.program_id(0) == 0)
    def _():
        esum_ref[...] = jnp.zeros_like(esum_ref)

    esum_ref[...] += jnp.sum(out, axis=0, keepdims=True)


def _edge_mlp(g, w0g, b0, w0e, w1, b1, w2, b2, edges, gsum):
    fixed = lambda shape: pl.BlockSpec(shape, lambda i: (0, 0))
    return pl.pallas_call(
        _edge_body,
        grid=(E // BE,),
        in_specs=[
            fixed((1, DN)), fixed((DN, DN)), fixed((1, DN)),
            fixed((DE, DN)), fixed((DN, DN)), fixed((1, DN)),
            fixed((DN, DN)), fixed((1, DN)),
            pl.BlockSpec((BE, DE), lambda i: (i, 0)),
            pl.BlockSpec((BE, DN), lambda i: (i, 0)),
        ],
        out_specs=(
            pl.BlockSpec((BE, DN), lambda i: (i, 0)),
            pl.BlockSpec((1, DN), lambda i: (0, 0)),
        ),
        out_shape=(
            jax.ShapeDtypeStruct((E, DN), jnp.float32),
            jax.ShapeDtypeStruct((1, DN), jnp.float32),
        ),
    )(g, w0g, b0, w0e, w1, b1, w2, b2, edges, gsum)


_NZC = RPT // CH
_NZC_LAST = (N - (NS - 1) * RPT) // CH


SSLOT = 3


@functools.partial(
    pl.kernel,
    out_type=jax.ShapeDtypeStruct((NC * N, DN), jnp.float32),
    mesh=_mesh,
    scratch_types=(
        [pltpu.VMEM((NCHUNK, CH), jnp.int32)]
        + [pltpu.VMEM((CH, DN), jnp.float32)] * SSLOT
        + [pltpu.SemaphoreType.DMA] * (2 * SSLOT)
        + [pltpu.VMEM_SHARED((N, DN), jnp.float32)]
    ),
)
def _scatter(eo_hbm, rcv_hbm, zrow_hbm, seg_hbm, *refs):
    ridx = refs[0]
    rows = refs[1:1 + SSLOT]
    sems = refs[1 + SSLOT:1 + 3 * SSLOT]
    rd, sa = sems[0:SSLOT], sems[SSLOT:2 * SSLOT]
    acc = refs[-1]

    cid = lax.axis_index("c")
    sid = lax.axis_index("s")
    wid = sid * NC + cid
    r0 = sid * RPT
    base0 = wid * EPW

    def _zero_acc():
        pltpu.sync_copy(zrow_hbm, rows[0])
        _ranged(sid, lambda k: pltpu.sync_copy(rows[0], acc.at[pl.ds(r0 + k * CH, CH)]))

    def _writeback(out_hbm):
        def fn(k):
            src = r0 + k * CH
            pltpu.sync_copy(acc.at[pl.ds(src, CH)], rows[0])
            pltpu.sync_copy(rows[0], out_hbm.at[pl.ds(cid * N + src, CH)])

        _ranged(sid, fn)

    pltpu.sync_copy(rcv_hbm.at[wid], ridx)
    _zero_acc()
    plsc.subcore_barrier()

    def rstart(j, b):
        pltpu.async_copy(eo_hbm.at[pl.ds(base0 + j * CH, CH)], rows[b], rd[b])

    def rwait(b):
        pltpu.make_async_copy(eo_hbm.at[pl.ds(0, CH)], rows[b], rd[b]).wait()

    def sstart(j, b):
        pltpu.async_copy(rows[b], acc.at[ridx.at[j]], sa[b], add=True)

    def sdrain(b):
        pltpu.make_async_copy(rows[b], acc.at[ridx.at[0]], sa[b]).wait()

    for b in range(SSLOT):
        rstart(b, b)

    def body(i, carry):
        for b in range(SSLOT):
            b1 = (b - 1) % SSLOT

            @pl.when(lax.rem(i, SSLOT) == b)
            def _():
                rwait(b)
                sstart(i, b)

                @pl.when((i >= 1) & (i - 1 + SSLOT < NCHUNK))
                def _():
                    sdrain(b1)
                    rstart(i - 1 + SSLOT, b1)

        return carry

    lax.fori_loop(0, NCHUNK, body, 0)
    for b in range(SSLOT):
        sdrain(b)
    plsc.subcore_barrier()
    _writeback(seg_hbm)


@functools.partial(
    pl.kernel,
    out_type=jax.ShapeDtypeStruct((NC * N, DN), jnp.float32),
    mesh=_mesh,
    scratch_types=(
        [pltpu.VMEM((NCHUNK, CH), jnp.int32)]
        + [pltpu.VMEM((CH, DN), jnp.float32)]
        + [pltpu.SemaphoreType.DMA] * SSLOT
        + [pltpu.VMEM_SHARED((N, DN), jnp.float32)]
    ),
)
def _counts(rcv_hbm, zrow_hbm, ones_hbm, cnt_hbm, *refs):
    ridx = refs[0]
    buf = refs[1]
    sa = refs[2:2 + SSLOT]
    acc = refs[-1]

    cid = lax.axis_index("c")
    sid = lax.axis_index("s")
    wid = sid * NC + cid
    r0 = sid * RPT

    pltpu.sync_copy(rcv_hbm.at[wid], ridx)
    pltpu.sync_copy(zrow_hbm, buf)
    _ranged(sid, lambda k: pltpu.sync_copy(buf, acc.at[pl.ds(r0 + k * CH, CH)]))
    plsc.subcore_barrier()
    pltpu.sync_copy(ones_hbm, buf)

    def cstart(j, b):
        pltpu.async_copy(buf, acc.at[ridx.at[j]], sa[b], add=True)

    def cdrain(b):
        pltpu.make_async_copy(buf, acc.at[ridx.at[0]], sa[b]).wait()

    for b in range(SSLOT):
        cstart(b, b)

    def body(i, carry):
        for b in range(SSLOT):
            @pl.when(lax.rem(i, SSLOT) == b)
            def _():
                cdrain(b)

                @pl.when(i + SSLOT < NCHUNK)
                def _():
                    cstart(i + SSLOT, b)

        return carry

    lax.fori_loop(0, NCHUNK, body, 0)
    plsc.subcore_barrier()

    def wb(k):
        src = r0 + k * CH
        pltpu.sync_copy(acc.at[pl.ds(src, CH)], buf)
        pltpu.sync_copy(buf, cnt_hbm.at[pl.ds(cid * N + src, CH)])

    _ranged(sid, wb)


def _node_glob_body(nodes_ref, seg_ref, cnt_ref, g_ref,
                    nw0a_ref, nw0b_ref, nw0c_ref, nb0_ref,
                    nw1_ref, nb1_ref, nw2_ref, nb2_ref,
                    gw0a_ref, gw0b_ref, gw0c_ref, gb0_ref,
                    gw1_ref, gb1_ref, gw2_ref, gb2_ref, esum_ref,
                    nout_ref, gout_ref):
    seg = seg_ref[0] + seg_ref[1]
    cnt = cnt_ref[0, :, 0:1] + cnt_ref[1, :, 0:1]
    agg = seg / jnp.maximum(cnt, 1.0)
    g = g_ref[...]
    gterm = jnp.dot(g, nw0c_ref[...], preferred_element_type=jnp.float32) + nb0_ref[...]
    h0 = jnp.maximum(
        jnp.dot(nodes_ref[...], nw0a_ref[...], preferred_element_type=jnp.float32)
        + jnp.dot(agg, nw0b_ref[...], preferred_element_type=jnp.float32)
        + gterm, 0.0)
    h1 = jnp.maximum(
        jnp.dot(h0, nw1_ref[...], preferred_element_type=jnp.float32)
        + nb1_ref[...], 0.0)
    nout = jnp.dot(h1, nw2_ref[...], preferred_element_type=jnp.float32) + nb2_ref[...]
    nout_ref[...] = nout

    nmean = jnp.sum(nout, axis=0, keepdims=True) * (1.0 / N)
    emean = esum_ref[...] * (1.0 / E)
    x = jnp.maximum(
        jnp.dot(g, gw0a_ref[...], preferred_element_type=jnp.float32)
        + jnp.dot(nmean, gw0b_ref[...], preferred_element_type=jnp.float32)
        + jnp.dot(emean, gw0c_ref[...], preferred_element_type=jnp.float32)
        + gb0_ref[...], 0.0)
    x = jnp.maximum(
        jnp.dot(x, gw1_ref[...], preferred_element_type=jnp.float32)
        + gb1_ref[...], 0.0)
    gout_ref[...] = jnp.dot(x, gw2_ref[...], preferred_element_type=jnp.float32) + gb2_ref[...]


def _node_glob(nodes, seg, cnt, g, nw0a, nw0b, nw0c, nb0, nw1, nb1, nw2, nb2,
               gw0a, gw0b, gw0c, gb0, gw1, gb1, gw2, gb2, esum):
    return pl.pallas_call(
        _node_glob_body,
        out_shape=(
            jax.ShapeDtypeStruct((N, DN), jnp.float32),
            jax.ShapeDtypeStruct((1, DN), jnp.float32),
        ),
    )(nodes, seg, cnt, g, nw0a, nw0b, nw0c, nb0, nw1, nb1, nw2, nb2,
      gw0a, gw0b, gw0c, gb0, gw1, gb1, gw2, gb2, esum)


def kernel(nodes, edges, global_attr, senders, receivers,
           eW0, eb0, eW1, eb1, eW2, eb2,
           nW0, nb0, nW1, nb1, nW2, nb2,
           gW0, gb0, gW1, gb1, gW2, gb2):
    senders = senders.astype(jnp.int32)
    receivers = receivers.astype(jnp.int32)
    row = lambda b: b.reshape(1, -1)

    snd3 = senders.reshape(NW, NCHUNK, CH)
    rcv3 = receivers.reshape(NW, NCHUNK, CH)
    zrow = jnp.zeros((CH, DN), jnp.float32)
    ones = jnp.ones((CH, DN), jnp.float32)

    bf = jnp.bfloat16
    ps, pr = _proj(nodes, eW0[DE:DE + DN], eW0[DE + DN:DE + 2 * DN])
    gsum = _gather(ps, pr, snd3, rcv3)
    cnt = _counts(rcv3, zrow, ones)
    edges_out, esum = _edge_mlp(
        global_attr, eW0[DE + 2 * DN:], row(eb0), eW0[:DE].astype(bf),
        eW1.astype(bf), row(eb1), eW2.astype(bf), row(eb2),
        edges.astype(bf), gsum)

    seg = _scatter(edges_out, rcv3, zrow)
    seg = seg.reshape(NC, N, DN)
    cnt = cnt.reshape(NC, N, DN)

    nodes_out, global_out = _node_glob(
        nodes, seg, cnt, global_attr,
        nW0[:DN], nW0[DN:2 * DN], nW0[2 * DN:], row(nb0),
        nW1, row(nb1), nW2, row(nb2),
        gW0[:DN], gW0[DN:2 * DN], gW0[2 * DN:], row(gb0),
        gW1, row(gb1), gW2, row(gb2), esum)
    return (nodes_out, edges_out, global_out)


# speedup vs baseline: 1.0018x; 1.0018x over previous
import functools

import jax
import jax.numpy as jnp
from jax import lax
from jax.experimental import pallas as pl
from jax.experimental.pallas import tpu as pltpu
from jax.experimental.pallas import tpu_sc as plsc

N = 10000
E = 320000
DN = 128
DE = 16

NC = 2
NS = 16
NW = NC * NS
EPW = E // NW
CH = 80
NCHUNK = EPW // CH
RPT = 640

_mesh = plsc.VectorSubcoreMesh(core_axis_name="c", subcore_axis_name="s")


def _proj_body(nodes_ref, ws_ref, wr_ref, ps_ref, pr_ref):
    n = nodes_ref[...]
    ps_ref[...] = jnp.dot(n, ws_ref[...], preferred_element_type=jnp.float32)
    pr_ref[...] = jnp.dot(n, wr_ref[...], preferred_element_type=jnp.float32)


def _proj(nodes, ws, wr):
    return pl.pallas_call(
        _proj_body,
        out_shape=(
            jax.ShapeDtypeStruct((N, DN), jnp.float32),
            jax.ShapeDtypeStruct((N, DN), jnp.float32),
        ),
    )(nodes, ws, wr)


NSLOT = 6
_NZC = RPT // CH
_NZC_LAST = (N - (NS - 1) * RPT) // CH


def _ranged(sid, fn):
    @pl.when(sid < NS - 1)
    def _():
        for k in range(_NZC):
            fn(k)

    @pl.when(sid == NS - 1)
    def _():
        for k in range(_NZC_LAST):
            fn(k)


@functools.partial(
    pl.kernel,
    out_type=jax.ShapeDtypeStruct((E, DN), jnp.float32),
    mesh=_mesh,
    scratch_types=(
        [pltpu.VMEM((NCHUNK, CH), jnp.int32)] * 2
        + [pltpu.VMEM((CH, DN), jnp.float32)] * NSLOT
        + [pltpu.SemaphoreType.DMA] * (3 * NSLOT)
    ),
)
def _gather(ps_hbm, pr_hbm, snd_hbm, rcv_hbm, gsum_hbm, *refs):
    sidx, ridx = refs[0], refs[1]
    buf = refs[2:2 + NSLOT]
    sems = refs[2 + NSLOT:]
    ga, gb = sems[0:NSLOT], sems[NSLOT:2 * NSLOT]
    wa = sems[2 * NSLOT:3 * NSLOT]

    wid = lax.axis_index("s") * NC + lax.axis_index("c")
    base0 = wid * EPW

    pltpu.sync_copy(snd_hbm.at[wid], sidx)
    pltpu.sync_copy(rcv_hbm.at[wid], ridx)

    def gstart(j, b):
        pltpu.async_copy(ps_hbm.at[sidx.at[j]], buf[b], ga[b])

    def gwait(b):
        pltpu.make_async_copy(ps_hbm.at[sidx.at[0]], buf[b], ga[b]).wait()

    def astart(j, b):
        pltpu.async_copy(pr_hbm.at[ridx.at[j]], buf[b], gb[b], add=True)

    def await_(b):
        pltpu.make_async_copy(pr_hbm.at[ridx.at[0]], buf[b], gb[b]).wait()

    def wstart(j, b):
        pltpu.async_copy(buf[b], gsum_hbm.at[pl.ds(base0 + j * CH, CH)], wa[b])

    def wdrain(b):
        pltpu.make_async_copy(buf[b], gsum_hbm.at[pl.ds(0, CH)], wa[b]).wait()

    for b in range(NSLOT):
        gstart(b, b)

    def body(i, carry):
        for b in range(NSLOT):
            b1 = (b - 1) % NSLOT
            b2 = (b - 2) % NSLOT

            @pl.when(lax.rem(i, NSLOT) == b)
            def _():
                gwait(b)
                astart(i, b)

                @pl.when(i >= 1)
                def _():
                    await_(b1)
                    wstart(i - 1, b1)

                @pl.when((i >= 2) & (i - 2 + NSLOT < NCHUNK))
                def _():
                    wdrain(b2)
                    gstart(i - 2 + NSLOT, b2)

        return carry

    lax.fori_loop(0, NCHUNK, body, 0)
    bl = (NCHUNK - 1) % NSLOT
    await_(bl)
    wstart(NCHUNK - 1, bl)
    wdrain((NCHUNK - 2) % NSLOT)
    wdrain(bl)


BE = 4000


def _edge_body(g_ref, w0g_ref, b0_ref, w0e_ref, w1_ref, b1_ref, w2_ref, b2_ref,
               e_ref, gsum_ref, out_ref, esum_ref):
    gvec = jnp.dot(g_ref[...], w0g_ref[...],
                   preferred_element_type=jnp.float32) + b0_ref[...]
    h0 = jnp.dot(e_ref[...], w0e_ref[...], preferred_element_type=jnp.float32)
    h0 = jnp.maximum(h0 + gsum_ref[...] + gvec, 0.0)
    h1 = jnp.maximum(
        jnp.dot(h0.astype(jnp.bfloat16), w1_ref[...],
                preferred_element_type=jnp.float32) + b1_ref[...], 0.0)
    out = jnp.dot(h1.astype(jnp.bfloat16), w2_ref[...],
                  preferred_element_type=jnp.float32) + b2_ref[...]
    out_ref[...] = out

    @pl.when(pl.program_id(0) == 0)
    def _():
        esum_ref[...] = jnp.zeros_like(esum_ref)

    esum_ref[...] += jnp.sum(out, axis=0, keepdims=True)


def _edge_mlp(g, w0g, b0, w0e, w1, b1, w2, b2, edges, gsum):
    fixed = lambda shape: pl.BlockSpec(shape, lambda i: (0, 0))
    return pl.pallas_call(
        _edge_body,
        grid=(E // BE,),
        in_specs=[
            fixed((1, DN)), fixed((DN, DN)), fixed((1, DN)),
            fixed((DE, DN)), fixed((DN, DN)), fixed((1, DN)),
            fixed((DN, DN)), fixed((1, DN)),
            pl.BlockSpec((BE, DE), lambda i: (i, 0)),
            pl.BlockSpec((BE, DN), lambda i: (i, 0)),
        ],
        out_specs=(
            pl.BlockSpec((BE, DN), lambda i: (i, 0)),
            pl.BlockSpec((1, DN), lambda i: (0, 0)),
        ),
        out_shape=(
            jax.ShapeDtypeStruct((E, DN), jnp.float32),
            jax.ShapeDtypeStruct((1, DN), jnp.float32),
        ),
    )(g, w0g, b0, w0e, w1, b1, w2, b2, edges, gsum)


_NZC = RPT // CH
_NZC_LAST = (N - (NS - 1) * RPT) // CH


SSLOT = 3


@functools.partial(
    pl.kernel,
    out_type=jax.ShapeDtypeStruct((NC * N, DN), jnp.float32),
    mesh=_mesh,
    scratch_types=(
        [pltpu.VMEM((NCHUNK, CH), jnp.int32)]
        + [pltpu.VMEM((CH, DN), jnp.float32)] * SSLOT
        + [pltpu.SemaphoreType.DMA] * (2 * SSLOT)
        + [pltpu.VMEM_SHARED((N, DN), jnp.float32)]
    ),
)
def _scatter(eo_hbm, rcv_hbm, zrow_hbm, seg_hbm, *refs):
    ridx = refs[0]
    rows = refs[1:1 + SSLOT]
    sems = refs[1 + SSLOT:1 + 3 * SSLOT]
    rd, sa = sems[0:SSLOT], sems[SSLOT:2 * SSLOT]
    acc = refs[-1]

    cid = lax.axis_index("c")
    sid = lax.axis_index("s")
    wid = sid * NC + cid
    r0 = sid * RPT
    base0 = wid * EPW

    def _zero_acc():
        pltpu.sync_copy(zrow_hbm, rows[0])
        _ranged(sid, lambda k: pltpu.sync_copy(rows[0], acc.at[pl.ds(r0 + k * CH, CH)]))

    def _writeback(out_hbm):
        def fn(k):
            src = r0 + k * CH
            pltpu.sync_copy(acc.at[pl.ds(src, CH)], rows[0])
            pltpu.sync_copy(rows[0], out_hbm.at[pl.ds(cid * N + src, CH)])

        _ranged(sid, fn)

    pltpu.sync_copy(rcv_hbm.at[wid], ridx)
    _zero_acc()
    plsc.subcore_barrier()

    def rstart(j, b):
        pltpu.async_copy(eo_hbm.at[pl.ds(base0 + j * CH, CH)], rows[b], rd[b])

    def rwait(b):
        pltpu.make_async_copy(eo_hbm.at[pl.ds(0, CH)], rows[b], rd[b]).wait()

    def sstart(j, b):
        pltpu.async_copy(rows[b], acc.at[ridx.at[j]], sa[b], add=True)

    def sdrain(b):
        pltpu.make_async_copy(rows[b], acc.at[ridx.at[0]], sa[b]).wait()

    for b in range(SSLOT):
        rstart(b, b)

    def body(i, carry):
        for b in range(SSLOT):
            b1 = (b - 1) % SSLOT

            @pl.when(lax.rem(i, SSLOT) == b)
            def _():
                rwait(b)
                sstart(i, b)

                @pl.when((i >= 1) & (i - 1 + SSLOT < NCHUNK))
                def _():
                    sdrain(b1)
                    rstart(i - 1 + SSLOT, b1)

        return carry

    lax.fori_loop(0, NCHUNK, body, 0)
    for b in range(SSLOT):
        sdrain(b)
    plsc.subcore_barrier()
    _writeback(seg_hbm)


@functools.partial(
    pl.kernel,
    out_type=jax.ShapeDtypeStruct((NC * N, DN), jnp.float32),
    mesh=_mesh,
    scratch_types=(
        [pltpu.VMEM((NCHUNK, CH), jnp.int32)]
        + [pltpu.VMEM((CH, DN), jnp.float32)]
        + [pltpu.SemaphoreType.DMA] * SSLOT
        + [pltpu.VMEM_SHARED((N, DN), jnp.float32)]
    ),
)
def _counts(rcv_hbm, zrow_hbm, ones_hbm, cnt_hbm, *refs):
    ridx = refs[0]
    buf = refs[1]
    sa = refs[2:2 + SSLOT]
    acc = refs[-1]

    cid = lax.axis_index("c")
    sid = lax.axis_index("s")
    wid = sid * NC + cid
    r0 = sid * RPT

    pltpu.sync_copy(rcv_hbm.at[wid], ridx)
    pltpu.sync_copy(zrow_hbm, buf)
    _ranged(sid, lambda k: pltpu.sync_copy(buf, acc.at[pl.ds(r0 + k * CH, CH)]))
    plsc.subcore_barrier()
    pltpu.sync_copy(ones_hbm, buf)

    def cstart(j, b):
        pltpu.async_copy(buf, acc.at[ridx.at[j]], sa[b], add=True)

    def cdrain(b):
        pltpu.make_async_copy(buf, acc.at[ridx.at[0]], sa[b]).wait()

    for b in range(SSLOT):
        cstart(b, b)

    def body(i, carry):
        for b in range(SSLOT):
            @pl.when(lax.rem(i, SSLOT) == b)
            def _():
                cdrain(b)

                @pl.when(i + SSLOT < NCHUNK)
                def _():
                    cstart(i + SSLOT, b)

        return carry

    lax.fori_loop(0, NCHUNK, body, 0)
    plsc.subcore_barrier()

    def wb(k):
        src = r0 + k * CH
        pltpu.sync_copy(acc.at[pl.ds(src, CH)], buf)
        pltpu.sync_copy(buf, cnt_hbm.at[pl.ds(cid * N + src, CH)])

    _ranged(sid, wb)


def _node_glob_body(nodes_ref, seg_ref, cnt_ref, g_ref,
                    nw0a_ref, nw0b_ref, nw0c_ref, nb0_ref,
                    nw1_ref, nb1_ref, nw2_ref, nb2_ref,
                    gw0a_ref, gw0b_ref, gw0c_ref, gb0_ref,
                    gw1_ref, gb1_ref, gw2_ref, gb2_ref, esum_ref,
                    nout_ref, gout_ref):
    seg = seg_ref[0] + seg_ref[1]
    cnt = cnt_ref[0, :, 0:1] + cnt_ref[1, :, 0:1]
    agg = seg / jnp.maximum(cnt, 1.0)
    g = g_ref[...]
    gterm = jnp.dot(g, nw0c_ref[...], preferred_element_type=jnp.float32) + nb0_ref[...]
    h0 = jnp.maximum(
        jnp.dot(nodes_ref[...], nw0a_ref[...], preferred_element_type=jnp.float32)
        + jnp.dot(agg, nw0b_ref[...], preferred_element_type=jnp.float32)
        + gterm, 0.0)
    h1 = jnp.maximum(
        jnp.dot(h0, nw1_ref[...], preferred_element_type=jnp.float32)
        + nb1_ref[...], 0.0)
    nout = jnp.dot(h1, nw2_ref[...], preferred_element_type=jnp.float32) + nb2_ref[...]
    nout_ref[...] = nout

    nmean = jnp.sum(nout, axis=0, keepdims=True) * (1.0 / N)
    emean = esum_ref[...] * (1.0 / E)
    x = jnp.maximum(
        jnp.dot(g, gw0a_ref[...], preferred_element_type=jnp.float32)
        + jnp.dot(nmean, gw0b_ref[...], preferred_element_type=jnp.float32)
        + jnp.dot(emean, gw0c_ref[...], preferred_element_type=jnp.float32)
        + gb0_ref[...], 0.0)
    x = jnp.maximum(
        jnp.dot(x, gw1_ref[...], preferred_element_type=jnp.float32)
        + gb1_ref[...], 0.0)
    gout_ref[...] = jnp.dot(x, gw2_ref[...], preferred_element_type=jnp.float32) + gb2_ref[...]


def _node_glob(nodes, seg, cnt, g, nw0a, nw0b, nw0c, nb0, nw1, nb1, nw2, nb2,
               gw0a, gw0b, gw0c, gb0, gw1, gb1, gw2, gb2, esum):
    return pl.pallas_call(
        _node_glob_body,
        out_shape=(
            jax.ShapeDtypeStruct((N, DN), jnp.float32),
            jax.ShapeDtypeStruct((1, DN), jnp.float32),
        ),
    )(nodes, seg, cnt, g, nw0a, nw0b, nw0c, nb0, nw1, nb1, nw2, nb2,
      gw0a, gw0b, gw0c, gb0, gw1, gb1, gw2, gb2, esum)


def kernel(nodes, edges, global_attr, senders, receivers,
           eW0, eb0, eW1, eb1, eW2, eb2,
           nW0, nb0, nW1, nb1, nW2, nb2,
           gW0, gb0, gW1, gb1, gW2, gb2):
    senders = senders.astype(jnp.int32)
    receivers = receivers.astype(jnp.int32)
    row = lambda b: b.reshape(1, -1)

    snd3 = senders.reshape(NW, NCHUNK, CH)
    rcv3 = receivers.reshape(NW, NCHUNK, CH)
    zrow = jnp.zeros((CH, DN), jnp.float32)
    ones = jnp.ones((CH, DN), jnp.float32)

    bf = jnp.bfloat16
    ps, pr = _proj(nodes, eW0[DE:DE + DN], eW0[DE + DN:DE + 2 * DN])
    gsum = _gather(ps, pr, snd3, rcv3)
    cnt = _counts(rcv3, zrow, ones)
    edges_out, esum = _edge_mlp(
        global_attr, eW0[DE + 2 * DN:], row(eb0), eW0[:DE].astype(bf),
        eW1.astype(bf), row(eb1), eW2.astype(bf), row(eb2),
        edges.astype(bf), gsum)

    seg = _scatter(edges_out, rcv3, zrow)
    seg = seg.reshape(NC, N, DN)
    cnt = cnt.reshape(NC, N, DN)

    nodes_out, global_out = _node_glob(
        nodes, seg, cnt, global_attr,
        nW0[:DN], nW0[DN:2 * DN], nW0[2 * DN:], row(nb0),
        nW1, row(nb1), nW2, row(nb2),
        gW0[:DN], gW0[DN:2 * DN], gW0[2 * DN:], row(gb0),
        gW1, row(gb1), gW2, row(gb2), esum)
    return (nodes_out, edges_out, global_out)


# speedup vs baseline: 1.0094x; 1.0076x over previous
import functools

import jax
import jax.numpy as jnp
from jax import lax
from jax.experimental import pallas as pl
from jax.experimental.pallas import tpu as pltpu
from jax.experimental.pallas import tpu_sc as plsc

N = 10000
E = 320000
DN = 128
DE = 16

NC = 2
NS = 16
NW = NC * NS
EPW = E // NW
CH = 80
NCHUNK = EPW // CH
RPT = 640

_mesh = plsc.VectorSubcoreMesh(core_axis_name="c", subcore_axis_name="s")


def _proj_body(nodes_ref, ws_ref, wr_ref, ps_ref, pr_ref):
    n = nodes_ref[...]
    ps_ref[...] = jnp.dot(n, ws_ref[...], preferred_element_type=jnp.float32)
    pr_ref[...] = jnp.dot(n, wr_ref[...], preferred_element_type=jnp.float32)


def _proj(nodes, ws, wr):
    return pl.pallas_call(
        _proj_body,
        out_shape=(
            jax.ShapeDtypeStruct((N, DN), jnp.float32),
            jax.ShapeDtypeStruct((N, DN), jnp.float32),
        ),
    )(nodes, ws, wr)


NSLOT = 6
_NZC = RPT // CH
_NZC_LAST = (N - (NS - 1) * RPT) // CH


def _ranged(sid, fn):
    @pl.when(sid < NS - 1)
    def _():
        for k in range(_NZC):
            fn(k)

    @pl.when(sid == NS - 1)
    def _():
        for k in range(_NZC_LAST):
            fn(k)


@functools.partial(
    pl.kernel,
    out_type=jax.ShapeDtypeStruct((E, DN), jnp.float32),
    mesh=_mesh,
    scratch_types=(
        [pltpu.VMEM((NCHUNK, CH), jnp.int32)] * 2
        + [pltpu.VMEM((CH, DN), jnp.float32)] * NSLOT
        + [pltpu.SemaphoreType.DMA] * (3 * NSLOT)
    ),
)
def _gather(ps_hbm, pr_hbm, snd_hbm, rcv_hbm, gsum_hbm, *refs):
    sidx, ridx = refs[0], refs[1]
    buf = refs[2:2 + NSLOT]
    sems = refs[2 + NSLOT:]
    ga, gb = sems[0:NSLOT], sems[NSLOT:2 * NSLOT]
    wa = sems[2 * NSLOT:3 * NSLOT]

    wid = lax.axis_index("s") * NC + lax.axis_index("c")
    base0 = wid * EPW

    pltpu.sync_copy(snd_hbm.at[wid], sidx)
    pltpu.sync_copy(rcv_hbm.at[wid], ridx)

    def gstart(j, b):
        pltpu.async_copy(ps_hbm.at[sidx.at[j]], buf[b], ga[b])

    def gwait(b):
        pltpu.make_async_copy(ps_hbm.at[sidx.at[0]], buf[b], ga[b]).wait()

    def astart(j, b):
        pltpu.async_copy(pr_hbm.at[ridx.at[j]], buf[b], gb[b], add=True)

    def await_(b):
        pltpu.make_async_copy(pr_hbm.at[ridx.at[0]], buf[b], gb[b]).wait()

    def wstart(j, b):
        pltpu.async_copy(buf[b], gsum_hbm.at[pl.ds(base0 + j * CH, CH)], wa[b])

    def wdrain(b):
        pltpu.make_async_copy(buf[b], gsum_hbm.at[pl.ds(0, CH)], wa[b]).wait()

    for b in range(NSLOT):
        gstart(b, b)

    def body(i, carry):
        for b in range(NSLOT):
            b1 = (b - 1) % NSLOT
            b2 = (b - 2) % NSLOT

            @pl.when(lax.rem(i, NSLOT) == b)
            def _():
                gwait(b)
                astart(i, b)

                @pl.when(i >= 1)
                def _():
                    await_(b1)
                    wstart(i - 1, b1)

                @pl.when((i >= 2) & (i - 2 + NSLOT < NCHUNK))
                def _():
                    wdrain(b2)
                    gstart(i - 2 + NSLOT, b2)

        return carry

    lax.fori_loop(0, NCHUNK, body, 0)
    bl = (NCHUNK - 1) % NSLOT
    await_(bl)
    wstart(NCHUNK - 1, bl)
    wdrain((NCHUNK - 2) % NSLOT)
    wdrain(bl)


BE = 4000


def _edge_body(g_ref, w0g_ref, b0_ref, w0e_ref, w1_ref, b1_ref, w2_ref, b2_ref,
               e_ref, gsum_ref, out_ref, esum_ref):
    gvec = jnp.dot(g_ref[...], w0g_ref[...],
                   preferred_element_type=jnp.float32) + b0_ref[...]
    h0 = jnp.dot(e_ref[...], w0e_ref[...], preferred_element_type=jnp.float32)
    h0 = jnp.maximum(h0 + gsum_ref[...] + gvec, 0.0)
    h1 = jnp.maximum(
        jnp.dot(h0.astype(jnp.bfloat16), w1_ref[...],
                preferred_element_type=jnp.float32) + b1_ref[...], 0.0)
    out = jnp.dot(h1.astype(jnp.bfloat16), w2_ref[...],
                  preferred_element_type=jnp.float32) + b2_ref[...]
    out_ref[...] = out

    @pl.when(pl.program_id(0) == 0)
    def _():
        esum_ref[...] = jnp.zeros_like(esum_ref)

    esum_ref[...] += jnp.sum(out, axis=0, keepdims=True)


def _edge_mlp(g, w0g, b0, w0e, w1, b1, w2, b2, edges, gsum):
    fixed = lambda shape: pl.BlockSpec(shape, lambda i: (0, 0))
    return pl.pallas_call(
        _edge_body,
        grid=(E // BE,),
        in_specs=[
            fixed((1, DN)), fixed((DN, DN)), fixed((1, DN)),
            fixed((DE, DN)), fixed((DN, DN)), fixed((1, DN)),
            fixed((DN, DN)), fixed((1, DN)),
            pl.BlockSpec((BE, DE), lambda i: (i, 0)),
            pl.BlockSpec((BE, DN), lambda i: (i, 0)),
        ],
        out_specs=(
            pl.BlockSpec((BE, DN), lambda i: (i, 0)),
            pl.BlockSpec((1, DN), lambda i: (0, 0)),
        ),
        out_shape=(
            jax.ShapeDtypeStruct((E, DN), jnp.float32),
            jax.ShapeDtypeStruct((1, DN), jnp.float32),
        ),
    )(g, w0g, b0, w0e, w1, b1, w2, b2, edges, gsum)


_NZC = RPT // CH
_NZC_LAST = (N - (NS - 1) * RPT) // CH


SSLOT = 3


@functools.partial(
    pl.kernel,
    out_type=jax.ShapeDtypeStruct((NC * N, DN), jnp.float32),
    mesh=_mesh,
    scratch_types=(
        [pltpu.VMEM((NCHUNK, CH), jnp.int32)]
        + [pltpu.VMEM((CH, DN), jnp.float32)] * SSLOT
        + [pltpu.SemaphoreType.DMA] * (2 * SSLOT)
        + [pltpu.VMEM_SHARED((N, DN), jnp.float32)]
    ),
)
def _scatter(eo_hbm, rcv_hbm, zrow_hbm, seg_hbm, *refs):
    ridx = refs[0]
    rows = refs[1:1 + SSLOT]
    sems = refs[1 + SSLOT:1 + 3 * SSLOT]
    rd, sa = sems[0:SSLOT], sems[SSLOT:2 * SSLOT]
    acc = refs[-1]

    cid = lax.axis_index("c")
    sid = lax.axis_index("s")
    wid = sid * NC + cid
    r0 = sid * RPT
    base0 = wid * EPW

    def _zero_acc():
        pltpu.sync_copy(zrow_hbm, rows[0])
        _ranged(sid, lambda k: pltpu.sync_copy(rows[0], acc.at[pl.ds(r0 + k * CH, CH)]))

    def _writeback(out_hbm):
        def fn(k):
            src = r0 + k * CH
            pltpu.sync_copy(acc.at[pl.ds(src, CH)], rows[0])
            pltpu.sync_copy(rows[0], out_hbm.at[pl.ds(cid * N + src, CH)])

        _ranged(sid, fn)

    pltpu.sync_copy(rcv_hbm.at[wid], ridx)
    _zero_acc()
    plsc.subcore_barrier()

    def rstart(j, b):
        pltpu.async_copy(eo_hbm.at[pl.ds(base0 + j * CH, CH)], rows[b], rd[b])

    def rwait(b):
        pltpu.make_async_copy(eo_hbm.at[pl.ds(0, CH)], rows[b], rd[b]).wait()

    def sstart(j, b):
        pltpu.async_copy(rows[b], acc.at[ridx.at[j]], sa[b], add=True)

    def sdrain(b):
        pltpu.make_async_copy(rows[b], acc.at[ridx.at[0]], sa[b]).wait()

    for b in range(SSLOT):
        rstart(b, b)

    def body(i, carry):
        for b in range(SSLOT):
            @pl.when(lax.rem(i, SSLOT) == b)
            def _():
                rwait(b)
                sstart(i, b)

                @pl.when(i + SSLOT < NCHUNK)
                def _():
                    sdrain(b)
                    rstart(i + SSLOT, b)

        return carry

    lax.fori_loop(0, NCHUNK, body, 0)
    for b in range(SSLOT):
        sdrain(b)
    plsc.subcore_barrier()
    _writeback(seg_hbm)


@functools.partial(
    pl.kernel,
    out_type=jax.ShapeDtypeStruct((NC * N, DN), jnp.float32),
    mesh=_mesh,
    scratch_types=(
        [pltpu.VMEM((NCHUNK, CH), jnp.int32)]
        + [pltpu.VMEM((CH, DN), jnp.float32)]
        + [pltpu.SemaphoreType.DMA] * SSLOT
        + [pltpu.VMEM_SHARED((N, DN), jnp.float32)]
    ),
)
def _counts(rcv_hbm, zrow_hbm, ones_hbm, cnt_hbm, *refs):
    ridx = refs[0]
    buf = refs[1]
    sa = refs[2:2 + SSLOT]
    acc = refs[-1]

    cid = lax.axis_index("c")
    sid = lax.axis_index("s")
    wid = sid * NC + cid
    r0 = sid * RPT

    pltpu.sync_copy(rcv_hbm.at[wid], ridx)
    pltpu.sync_copy(zrow_hbm, buf)
    _ranged(sid, lambda k: pltpu.sync_copy(buf, acc.at[pl.ds(r0 + k * CH, CH)]))
    plsc.subcore_barrier()
    pltpu.sync_copy(ones_hbm, buf)

    def cstart(j, b):
        pltpu.async_copy(buf, acc.at[ridx.at[j]], sa[b], add=True)

    def cdrain(b):
        pltpu.make_async_copy(buf, acc.at[ridx.at[0]], sa[b]).wait()

    for b in range(SSLOT):
        cstart(b, b)

    def body(i, carry):
        for b in range(SSLOT):
            @pl.when(lax.rem(i, SSLOT) == b)
            def _():
                cdrain(b)

                @pl.when(i + SSLOT < NCHUNK)
                def _():
                    cstart(i + SSLOT, b)

        return carry

    lax.fori_loop(0, NCHUNK, body, 0)
    plsc.subcore_barrier()

    def wb(k):
        src = r0 + k * CH
        pltpu.sync_copy(acc.at[pl.ds(src, CH)], buf)
        pltpu.sync_copy(buf, cnt_hbm.at[pl.ds(cid * N + src, CH)])

    _ranged(sid, wb)


def _node_glob_body(nodes_ref, seg_ref, cnt_ref, g_ref,
                    nw0a_ref, nw0b_ref, nw0c_ref, nb0_ref,
                    nw1_ref, nb1_ref, nw2_ref, nb2_ref,
                    gw0a_ref, gw0b_ref, gw0c_ref, gb0_ref,
                    gw1_ref, gb1_ref, gw2_ref, gb2_ref, esum_ref,
                    nout_ref, gout_ref):
    seg = seg_ref[0] + seg_ref[1]
    cnt = cnt_ref[0, :, 0:1] + cnt_ref[1, :, 0:1]
    agg = seg / jnp.maximum(cnt, 1.0)
    g = g_ref[...]
    gterm = jnp.dot(g, nw0c_ref[...], preferred_element_type=jnp.float32) + nb0_ref[...]
    h0 = jnp.maximum(
        jnp.dot(nodes_ref[...], nw0a_ref[...], preferred_element_type=jnp.float32)
        + jnp.dot(agg, nw0b_ref[...], preferred_element_type=jnp.float32)
        + gterm, 0.0)
    h1 = jnp.maximum(
        jnp.dot(h0, nw1_ref[...], preferred_element_type=jnp.float32)
        + nb1_ref[...], 0.0)
    nout = jnp.dot(h1, nw2_ref[...], preferred_element_type=jnp.float32) + nb2_ref[...]
    nout_ref[...] = nout

    nmean = jnp.sum(nout, axis=0, keepdims=True) * (1.0 / N)
    emean = esum_ref[...] * (1.0 / E)
    x = jnp.maximum(
        jnp.dot(g, gw0a_ref[...], preferred_element_type=jnp.float32)
        + jnp.dot(nmean, gw0b_ref[...], preferred_element_type=jnp.float32)
        + jnp.dot(emean, gw0c_ref[...], preferred_element_type=jnp.float32)
        + gb0_ref[...], 0.0)
    x = jnp.maximum(
        jnp.dot(x, gw1_ref[...], preferred_element_type=jnp.float32)
        + gb1_ref[...], 0.0)
    gout_ref[...] = jnp.dot(x, gw2_ref[...], preferred_element_type=jnp.float32) + gb2_ref[...]


def _node_glob(nodes, seg, cnt, g, nw0a, nw0b, nw0c, nb0, nw1, nb1, nw2, nb2,
               gw0a, gw0b, gw0c, gb0, gw1, gb1, gw2, gb2, esum):
    return pl.pallas_call(
        _node_glob_body,
        out_shape=(
            jax.ShapeDtypeStruct((N, DN), jnp.float32),
            jax.ShapeDtypeStruct((1, DN), jnp.float32),
        ),
    )(nodes, seg, cnt, g, nw0a, nw0b, nw0c, nb0, nw1, nb1, nw2, nb2,
      gw0a, gw0b, gw0c, gb0, gw1, gb1, gw2, gb2, esum)


def kernel(nodes, edges, global_attr, senders, receivers,
           eW0, eb0, eW1, eb1, eW2, eb2,
           nW0, nb0, nW1, nb1, nW2, nb2,
           gW0, gb0, gW1, gb1, gW2, gb2):
    senders = senders.astype(jnp.int32)
    receivers = receivers.astype(jnp.int32)
    row = lambda b: b.reshape(1, -1)

    snd3 = senders.reshape(NW, NCHUNK, CH)
    rcv3 = receivers.reshape(NW, NCHUNK, CH)
    zrow = jnp.zeros((CH, DN), jnp.float32)
    ones = jnp.ones((CH, DN), jnp.float32)

    bf = jnp.bfloat16
    ps, pr = _proj(nodes, eW0[DE:DE + DN], eW0[DE + DN:DE + 2 * DN])
    gsum = _gather(ps, pr, snd3, rcv3)
    cnt = _counts(rcv3, zrow, ones)
    edges_out, esum = _edge_mlp(
        global_attr, eW0[DE + 2 * DN:], row(eb0), eW0[:DE].astype(bf),
        eW1.astype(bf), row(eb1), eW2.astype(bf), row(eb2),
        edges.astype(bf), gsum)

    seg = _scatter(edges_out, rcv3, zrow)
    seg = seg.reshape(NC, N, DN)
    cnt = cnt.reshape(NC, N, DN)

    nodes_out, global_out = _node_glob(
        nodes, seg, cnt, global_attr,
        nW0[:DN], nW0[DN:2 * DN], nW0[2 * DN:], row(nb0),
        nW1, row(nb1), nW2, row(nb2),
        gW0[:DN], gW0[DN:2 * DN], gW0[2 * DN:], row(gb0),
        gW1, row(gb1), gW2, row(gb2), esum)
    return (nodes_out, edges_out, global_out)


# speedup vs baseline: 1.0628x; 1.0530x over previous
import functools

import jax
import jax.numpy as jnp
from jax import lax
from jax.experimental import pallas as pl
from jax.experimental.pallas import tpu as pltpu
from jax.experimental.pallas import tpu_sc as plsc

N = 10000
E = 320000
DN = 128
DE = 16

NC = 2
NS = 16
NW = NC * NS
EPW = E // NW
CH = 80
NCHUNK = EPW // CH
RPT = 640

_mesh = plsc.VectorSubcoreMesh(core_axis_name="c", subcore_axis_name="s")


def _proj_body(nodes_ref, ws_ref, wr_ref, ps_ref, pr_ref):
    n = nodes_ref[...]
    ps_ref[...] = jnp.dot(n, ws_ref[...], preferred_element_type=jnp.float32)
    pr_ref[...] = jnp.dot(n, wr_ref[...], preferred_element_type=jnp.float32)


def _proj(nodes, ws, wr):
    return pl.pallas_call(
        _proj_body,
        out_shape=(
            jax.ShapeDtypeStruct((N, DN), jnp.float32),
            jax.ShapeDtypeStruct((N, DN), jnp.float32),
        ),
    )(nodes, ws, wr)


NSLOT = 6
_NZC = RPT // CH
_NZC_LAST = (N - (NS - 1) * RPT) // CH


def _ranged(sid, fn):
    @pl.when(sid < NS - 1)
    def _():
        for k in range(_NZC):
            fn(k)

    @pl.when(sid == NS - 1)
    def _():
        for k in range(_NZC_LAST):
            fn(k)


@functools.partial(
    pl.kernel,
    out_type=jax.ShapeDtypeStruct((E, DN), jnp.float32),
    mesh=_mesh,
    scratch_types=(
        [pltpu.VMEM((NCHUNK, CH), jnp.int32)] * 2
        + [pltpu.VMEM((CH, DN), jnp.float32)] * NSLOT
        + [pltpu.SemaphoreType.DMA] * (3 * NSLOT)
    ),
)
def _gather(ps_hbm, pr_hbm, snd_hbm, rcv_hbm, gsum_hbm, *refs):
    sidx, ridx = refs[0], refs[1]
    buf = refs[2:2 + NSLOT]
    sems = refs[2 + NSLOT:]
    ga, gb = sems[0:NSLOT], sems[NSLOT:2 * NSLOT]
    wa = sems[2 * NSLOT:3 * NSLOT]

    wid = lax.axis_index("s") * NC + lax.axis_index("c")
    base0 = wid * EPW

    pltpu.sync_copy(snd_hbm.at[wid], sidx)
    pltpu.sync_copy(rcv_hbm.at[wid], ridx)

    def gstart(j, b):
        pltpu.async_copy(ps_hbm.at[sidx.at[j]], buf[b], ga[b])

    def gwait(b):
        pltpu.make_async_copy(ps_hbm.at[sidx.at[0]], buf[b], ga[b]).wait()

    def astart(j, b):
        pltpu.async_copy(pr_hbm.at[ridx.at[j]], buf[b], gb[b], add=True)

    def await_(b):
        pltpu.make_async_copy(pr_hbm.at[ridx.at[0]], buf[b], gb[b]).wait()

    def wstart(j, b):
        pltpu.async_copy(buf[b], gsum_hbm.at[pl.ds(base0 + j * CH, CH)], wa[b])

    def wdrain(b):
        pltpu.make_async_copy(buf[b], gsum_hbm.at[pl.ds(0, CH)], wa[b]).wait()

    for b in range(NSLOT):
        gstart(b, b)

    def body(i, carry):
        for b in range(NSLOT):
            b1 = (b - 1) % NSLOT
            b2 = (b - 2) % NSLOT

            @pl.when(lax.rem(i, NSLOT) == b)
            def _():
                gwait(b)
                astart(i, b)

                @pl.when(i >= 1)
                def _():
                    await_(b1)
                    wstart(i - 1, b1)

                @pl.when((i >= 2) & (i - 2 + NSLOT < NCHUNK))
                def _():
                    wdrain(b2)
                    gstart(i - 2 + NSLOT, b2)

        return carry

    lax.fori_loop(0, NCHUNK, body, 0)
    bl = (NCHUNK - 1) % NSLOT
    await_(bl)
    wstart(NCHUNK - 1, bl)
    wdrain((NCHUNK - 2) % NSLOT)
    wdrain(bl)


BE = 8000


def _edge_body(g_ref, w0g_ref, b0_ref, w0e_ref, w1_ref, b1_ref, w2_ref, b2_ref,
               e_ref, gsum_ref, out_ref, esum_ref):
    gvec = jnp.dot(g_ref[...], w0g_ref[...],
                   preferred_element_type=jnp.float32) + b0_ref[...]
    h0 = jnp.dot(e_ref[...], w0e_ref[...], preferred_element_type=jnp.float32)
    h0 = jnp.maximum(h0 + gsum_ref[...] + gvec, 0.0)
    h1 = jnp.maximum(
        jnp.dot(h0.astype(jnp.bfloat16), w1_ref[...],
                preferred_element_type=jnp.float32) + b1_ref[...], 0.0)
    out = jnp.dot(h1.astype(jnp.bfloat16), w2_ref[...],
                  preferred_element_type=jnp.float32) + b2_ref[...]
    out_ref[...] = out

    @pl.when(pl.program_id(0) == 0)
    def _():
        esum_ref[...] = jnp.zeros_like(esum_ref)

    esum_ref[...] += jnp.sum(out, axis=0, keepdims=True)


def _edge_mlp(g, w0g, b0, w0e, w1, b1, w2, b2, edges, gsum):
    fixed = lambda shape: pl.BlockSpec(shape, lambda i: (0, 0))
    return pl.pallas_call(
        _edge_body,
        grid=(E // BE,),
        in_specs=[
            fixed((1, DN)), fixed((DN, DN)), fixed((1, DN)),
            fixed((DE, DN)), fixed((DN, DN)), fixed((1, DN)),
            fixed((DN, DN)), fixed((1, DN)),
            pl.BlockSpec((BE, DE), lambda i: (i, 0)),
            pl.BlockSpec((BE, DN), lambda i: (i, 0)),
        ],
        out_specs=(
            pl.BlockSpec((BE, DN), lambda i: (i, 0)),
            pl.BlockSpec((1, DN), lambda i: (0, 0)),
        ),
        out_shape=(
            jax.ShapeDtypeStruct((E, DN), jnp.float32),
            jax.ShapeDtypeStruct((1, DN), jnp.float32),
        ),
    )(g, w0g, b0, w0e, w1, b1, w2, b2, edges, gsum)


_NZC = RPT // CH
_NZC_LAST = (N - (NS - 1) * RPT) // CH


SSLOT = 3


@functools.partial(
    pl.kernel,
    out_type=jax.ShapeDtypeStruct((NC * N, DN), jnp.float32),
    mesh=_mesh,
    scratch_types=(
        [pltpu.VMEM((NCHUNK, CH), jnp.int32)]
        + [pltpu.VMEM((CH, DN), jnp.float32)] * SSLOT
        + [pltpu.SemaphoreType.DMA] * (2 * SSLOT)
        + [pltpu.VMEM_SHARED((N, DN), jnp.float32)]
    ),
)
def _scatter(eo_hbm, rcv_hbm, zrow_hbm, seg_hbm, *refs):
    ridx = refs[0]
    rows = refs[1:1 + SSLOT]
    sems = refs[1 + SSLOT:1 + 3 * SSLOT]
    rd, sa = sems[0:SSLOT], sems[SSLOT:2 * SSLOT]
    acc = refs[-1]

    cid = lax.axis_index("c")
    sid = lax.axis_index("s")
    wid = sid * NC + cid
    r0 = sid * RPT
    base0 = wid * EPW

    def _zero_acc():
        pltpu.sync_copy(zrow_hbm, rows[0])
        _ranged(sid, lambda k: pltpu.sync_copy(rows[0], acc.at[pl.ds(r0 + k * CH, CH)]))

    def _writeback(out_hbm):
        def fn(k):
            src = r0 + k * CH
            pltpu.sync_copy(acc.at[pl.ds(src, CH)], rows[0])
            pltpu.sync_copy(rows[0], out_hbm.at[pl.ds(cid * N + src, CH)])

        _ranged(sid, fn)

    pltpu.sync_copy(rcv_hbm.at[wid], ridx)
    _zero_acc()
    plsc.subcore_barrier()

    def rstart(j, b):
        pltpu.async_copy(eo_hbm.at[pl.ds(base0 + j * CH, CH)], rows[b], rd[b])

    def rwait(b):
        pltpu.make_async_copy(eo_hbm.at[pl.ds(0, CH)], rows[b], rd[b]).wait()

    def sstart(j, b):
        pltpu.async_copy(rows[b], acc.at[ridx.at[j]], sa[b], add=True)

    def sdrain(b):
        pltpu.make_async_copy(rows[b], acc.at[ridx.at[0]], sa[b]).wait()

    for b in range(SSLOT):
        rstart(b, b)

    def body(i, carry):
        for b in range(SSLOT):
            @pl.when(lax.rem(i, SSLOT) == b)
            def _():
                rwait(b)
                sstart(i, b)

                @pl.when(i + SSLOT < NCHUNK)
                def _():
                    sdrain(b)
                    rstart(i + SSLOT, b)

        return carry

    lax.fori_loop(0, NCHUNK, body, 0)
    for b in range(SSLOT):
        sdrain(b)
    plsc.subcore_barrier()
    _writeback(seg_hbm)


@functools.partial(
    pl.kernel,
    out_type=jax.ShapeDtypeStruct((NC * N, DN), jnp.float32),
    mesh=_mesh,
    scratch_types=(
        [pltpu.VMEM((NCHUNK, CH), jnp.int32)]
        + [pltpu.VMEM((CH, DN), jnp.float32)]
        + [pltpu.SemaphoreType.DMA] * SSLOT
        + [pltpu.VMEM_SHARED((N, DN), jnp.float32)]
    ),
)
def _counts(rcv_hbm, zrow_hbm, ones_hbm, cnt_hbm, *refs):
    ridx = refs[0]
    buf = refs[1]
    sa = refs[2:2 + SSLOT]
    acc = refs[-1]

    cid = lax.axis_index("c")
    sid = lax.axis_index("s")
    wid = sid * NC + cid
    r0 = sid * RPT

    pltpu.sync_copy(rcv_hbm.at[wid], ridx)
    pltpu.sync_copy(zrow_hbm, buf)
    _ranged(sid, lambda k: pltpu.sync_copy(buf, acc.at[pl.ds(r0 + k * CH, CH)]))
    plsc.subcore_barrier()
    pltpu.sync_copy(ones_hbm, buf)

    def cstart(j, b):
        pltpu.async_copy(buf, acc.at[ridx.at[j]], sa[b], add=True)

    def cdrain(b):
        pltpu.make_async_copy(buf, acc.at[ridx.at[0]], sa[b]).wait()

    for b in range(SSLOT):
        cstart(b, b)

    def body(i, carry):
        for b in range(SSLOT):
            @pl.when(lax.rem(i, SSLOT) == b)
            def _():
                cdrain(b)

                @pl.when(i + SSLOT < NCHUNK)
                def _():
                    cstart(i + SSLOT, b)

        return carry

    lax.fori_loop(0, NCHUNK, body, 0)
    plsc.subcore_barrier()

    def wb(k):
        src = r0 + k * CH
        pltpu.sync_copy(acc.at[pl.ds(src, CH)], buf)
        pltpu.sync_copy(buf, cnt_hbm.at[pl.ds(cid * N + src, CH)])

    _ranged(sid, wb)


def _node_glob_body(nodes_ref, seg_ref, cnt_ref, g_ref,
                    nw0a_ref, nw0b_ref, nw0c_ref, nb0_ref,
                    nw1_ref, nb1_ref, nw2_ref, nb2_ref,
                    gw0a_ref, gw0b_ref, gw0c_ref, gb0_ref,
                    gw1_ref, gb1_ref, gw2_ref, gb2_ref, esum_ref,
                    nout_ref, gout_ref):
    seg = seg_ref[0] + seg_ref[1]
    cnt = cnt_ref[0, :, 0:1] + cnt_ref[1, :, 0:1]
    agg = seg / jnp.maximum(cnt, 1.0)
    g = g_ref[...]
    gterm = jnp.dot(g, nw0c_ref[...], preferred_element_type=jnp.float32) + nb0_ref[...]
    h0 = jnp.maximum(
        jnp.dot(nodes_ref[...], nw0a_ref[...], preferred_element_type=jnp.float32)
        + jnp.dot(agg, nw0b_ref[...], preferred_element_type=jnp.float32)
        + gterm, 0.0)
    h1 = jnp.maximum(
        jnp.dot(h0, nw1_ref[...], preferred_element_type=jnp.float32)
        + nb1_ref[...], 0.0)
    nout = jnp.dot(h1, nw2_ref[...], preferred_element_type=jnp.float32) + nb2_ref[...]
    nout_ref[...] = nout

    nmean = jnp.sum(nout, axis=0, keepdims=True) * (1.0 / N)
    emean = esum_ref[...] * (1.0 / E)
    x = jnp.maximum(
        jnp.dot(g, gw0a_ref[...], preferred_element_type=jnp.float32)
        + jnp.dot(nmean, gw0b_ref[...], preferred_element_type=jnp.float32)
        + jnp.dot(emean, gw0c_ref[...], preferred_element_type=jnp.float32)
        + gb0_ref[...], 0.0)
    x = jnp.maximum(
        jnp.dot(x, gw1_ref[...], preferred_element_type=jnp.float32)
        + gb1_ref[...], 0.0)
    gout_ref[...] = jnp.dot(x, gw2_ref[...], preferred_element_type=jnp.float32) + gb2_ref[...]


def _node_glob(nodes, seg, cnt, g, nw0a, nw0b, nw0c, nb0, nw1, nb1, nw2, nb2,
               gw0a, gw0b, gw0c, gb0, gw1, gb1, gw2, gb2, esum):
    return pl.pallas_call(
        _node_glob_body,
        out_shape=(
            jax.ShapeDtypeStruct((N, DN), jnp.float32),
            jax.ShapeDtypeStruct((1, DN), jnp.float32),
        ),
    )(nodes, seg, cnt, g, nw0a, nw0b, nw0c, nb0, nw1, nb1, nw2, nb2,
      gw0a, gw0b, gw0c, gb0, gw1, gb1, gw2, gb2, esum)


def kernel(nodes, edges, global_attr, senders, receivers,
           eW0, eb0, eW1, eb1, eW2, eb2,
           nW0, nb0, nW1, nb1, nW2, nb2,
           gW0, gb0, gW1, gb1, gW2, gb2):
    senders = senders.astype(jnp.int32)
    receivers = receivers.astype(jnp.int32)
    row = lambda b: b.reshape(1, -1)

    snd3 = senders.reshape(NW, NCHUNK, CH)
    rcv3 = receivers.reshape(NW, NCHUNK, CH)
    zrow = jnp.zeros((CH, DN), jnp.float32)
    ones = jnp.ones((CH, DN), jnp.float32)

    bf = jnp.bfloat16
    ps, pr = _proj(nodes, eW0[DE:DE + DN], eW0[DE + DN:DE + 2 * DN])
    gsum = _gather(ps, pr, snd3, rcv3)
    cnt = _counts(rcv3, zrow, ones)
    edges_out, esum = _edge_mlp(
        global_attr, eW0[DE + 2 * DN:], row(eb0), eW0[:DE].astype(bf),
        eW1.astype(bf), row(eb1), eW2.astype(bf), row(eb2),
        edges.astype(bf), gsum)

    seg = _scatter(edges_out, rcv3, zrow)
    seg = seg.reshape(NC, N, DN)
    cnt = cnt.reshape(NC, N, DN)

    nodes_out, global_out = _node_glob(
        nodes, seg, cnt, global_attr,
        nW0[:DN], nW0[DN:2 * DN], nW0[2 * DN:], row(nb0),
        nW1, row(nb1), nW2, row(nb2),
        gW0[:DN], gW0[DN:2 * DN], gW0[2 * DN:], row(gb0),
        gW1, row(gb1), gW2, row(gb2), esum)
    return (nodes_out, edges_out, global_out)


# speedup vs baseline: 1.0973x; 1.0324x over previous
import functools

import jax
import jax.numpy as jnp
from jax import lax
from jax.experimental import pallas as pl
from jax.experimental.pallas import tpu as pltpu
from jax.experimental.pallas import tpu_sc as plsc

N = 10000
E = 320000
DN = 128
DE = 16

NC = 2
NS = 16
NW = NC * NS
EPW = E // NW
CH = 80
NCHUNK = EPW // CH
RPT = 640

_mesh = plsc.VectorSubcoreMesh(core_axis_name="c", subcore_axis_name="s")


def _proj_body(nodes_ref, ws_ref, wr_ref, ps_ref, pr_ref):
    n = nodes_ref[...]
    ps_ref[...] = jnp.dot(n, ws_ref[...], preferred_element_type=jnp.float32)
    pr_ref[...] = jnp.dot(n, wr_ref[...], preferred_element_type=jnp.float32)


def _proj(nodes, ws, wr):
    return pl.pallas_call(
        _proj_body,
        out_shape=(
            jax.ShapeDtypeStruct((N, DN), jnp.float32),
            jax.ShapeDtypeStruct((N, DN), jnp.float32),
        ),
    )(nodes, ws, wr)


NSLOT = 6
_NZC = RPT // CH
_NZC_LAST = (N - (NS - 1) * RPT) // CH


def _ranged(sid, fn):
    @pl.when(sid < NS - 1)
    def _():
        for k in range(_NZC):
            fn(k)

    @pl.when(sid == NS - 1)
    def _():
        for k in range(_NZC_LAST):
            fn(k)


@functools.partial(
    pl.kernel,
    out_type=jax.ShapeDtypeStruct((E, DN), jnp.float32),
    mesh=_mesh,
    scratch_types=(
        [pltpu.VMEM((NCHUNK, CH), jnp.int32)] * 2
        + [pltpu.VMEM((CH, DN), jnp.float32)] * NSLOT
        + [pltpu.SemaphoreType.DMA] * (3 * NSLOT)
    ),
)
def _gather(ps_hbm, pr_hbm, snd_hbm, rcv_hbm, gsum_hbm, *refs):
    sidx, ridx = refs[0], refs[1]
    buf = refs[2:2 + NSLOT]
    sems = refs[2 + NSLOT:]
    ga, gb = sems[0:NSLOT], sems[NSLOT:2 * NSLOT]
    wa = sems[2 * NSLOT:3 * NSLOT]

    wid = lax.axis_index("s") * NC + lax.axis_index("c")
    base0 = wid * EPW

    pltpu.sync_copy(snd_hbm.at[wid], sidx)
    pltpu.sync_copy(rcv_hbm.at[wid], ridx)

    def gstart(j, b):
        pltpu.async_copy(ps_hbm.at[sidx.at[j]], buf[b], ga[b])

    def gwait(b):
        pltpu.make_async_copy(ps_hbm.at[sidx.at[0]], buf[b], ga[b]).wait()

    def astart(j, b):
        pltpu.async_copy(pr_hbm.at[ridx.at[j]], buf[b], gb[b], add=True)

    def await_(b):
        pltpu.make_async_copy(pr_hbm.at[ridx.at[0]], buf[b], gb[b]).wait()

    def wstart(j, b):
        pltpu.async_copy(buf[b], gsum_hbm.at[pl.ds(base0 + j * CH, CH)], wa[b])

    def wdrain(b):
        pltpu.make_async_copy(buf[b], gsum_hbm.at[pl.ds(0, CH)], wa[b]).wait()

    for b in range(NSLOT):
        gstart(b, b)

    def body(i, carry):
        for b in range(NSLOT):
            b1 = (b - 1) % NSLOT
            b2 = (b - 2) % NSLOT

            @pl.when(lax.rem(i, NSLOT) == b)
            def _():
                gwait(b)
                astart(i, b)

                @pl.when(i >= 1)
                def _():
                    await_(b1)
                    wstart(i - 1, b1)

                @pl.when((i >= 2) & (i - 2 + NSLOT < NCHUNK))
                def _():
                    wdrain(b2)
                    gstart(i - 2 + NSLOT, b2)

        return carry

    lax.fori_loop(0, NCHUNK, body, 0)
    bl = (NCHUNK - 1) % NSLOT
    await_(bl)
    wstart(NCHUNK - 1, bl)
    wdrain((NCHUNK - 2) % NSLOT)
    wdrain(bl)


BE = 16000


def _edge_body(g_ref, w0g_ref, b0_ref, w0e_ref, w1_ref, b1_ref, w2_ref, b2_ref,
               e_ref, gsum_ref, out_ref, esum_ref):
    gvec = jnp.dot(g_ref[...], w0g_ref[...],
                   preferred_element_type=jnp.float32) + b0_ref[...]
    h0 = jnp.dot(e_ref[...], w0e_ref[...], preferred_element_type=jnp.float32)
    h0 = jnp.maximum(h0 + gsum_ref[...] + gvec, 0.0)
    h1 = jnp.maximum(
        jnp.dot(h0.astype(jnp.bfloat16), w1_ref[...],
                preferred_element_type=jnp.float32) + b1_ref[...], 0.0)
    out = jnp.dot(h1.astype(jnp.bfloat16), w2_ref[...],
                  preferred_element_type=jnp.float32) + b2_ref[...]
    out_ref[...] = out

    @pl.when(pl.program_id(0) == 0)
    def _():
        esum_ref[...] = jnp.zeros_like(esum_ref)

    esum_ref[...] += jnp.sum(out, axis=0, keepdims=True)


def _edge_mlp(g, w0g, b0, w0e, w1, b1, w2, b2, edges, gsum):
    fixed = lambda shape: pl.BlockSpec(shape, lambda i: (0, 0))
    return pl.pallas_call(
        _edge_body,
        grid=(E // BE,),
        in_specs=[
            fixed((1, DN)), fixed((DN, DN)), fixed((1, DN)),
            fixed((DE, DN)), fixed((DN, DN)), fixed((1, DN)),
            fixed((DN, DN)), fixed((1, DN)),
            pl.BlockSpec((BE, DE), lambda i: (i, 0)),
            pl.BlockSpec((BE, DN), lambda i: (i, 0)),
        ],
        out_specs=(
            pl.BlockSpec((BE, DN), lambda i: (i, 0)),
            pl.BlockSpec((1, DN), lambda i: (0, 0)),
        ),
        out_shape=(
            jax.ShapeDtypeStruct((E, DN), jnp.float32),
            jax.ShapeDtypeStruct((1, DN), jnp.float32),
        ),
    )(g, w0g, b0, w0e, w1, b1, w2, b2, edges, gsum)


_NZC = RPT // CH
_NZC_LAST = (N - (NS - 1) * RPT) // CH


SSLOT = 3


@functools.partial(
    pl.kernel,
    out_type=jax.ShapeDtypeStruct((NC * N, DN), jnp.float32),
    mesh=_mesh,
    scratch_types=(
        [pltpu.VMEM((NCHUNK, CH), jnp.int32)]
        + [pltpu.VMEM((CH, DN), jnp.float32)] * SSLOT
        + [pltpu.SemaphoreType.DMA] * (2 * SSLOT)
        + [pltpu.VMEM_SHARED((N, DN), jnp.float32)]
    ),
)
def _scatter(eo_hbm, rcv_hbm, zrow_hbm, seg_hbm, *refs):
    ridx = refs[0]
    rows = refs[1:1 + SSLOT]
    sems = refs[1 + SSLOT:1 + 3 * SSLOT]
    rd, sa = sems[0:SSLOT], sems[SSLOT:2 * SSLOT]
    acc = refs[-1]

    cid = lax.axis_index("c")
    sid = lax.axis_index("s")
    wid = sid * NC + cid
    r0 = sid * RPT
    base0 = wid * EPW

    def _zero_acc():
        pltpu.sync_copy(zrow_hbm, rows[0])
        _ranged(sid, lambda k: pltpu.sync_copy(rows[0], acc.at[pl.ds(r0 + k * CH, CH)]))

    def _writeback(out_hbm):
        def fn(k):
            src = r0 + k * CH
            pltpu.sync_copy(acc.at[pl.ds(src, CH)], rows[0])
            pltpu.sync_copy(rows[0], out_hbm.at[pl.ds(cid * N + src, CH)])

        _ranged(sid, fn)

    pltpu.sync_copy(rcv_hbm.at[wid], ridx)
    _zero_acc()
    plsc.subcore_barrier()

    def rstart(j, b):
        pltpu.async_copy(eo_hbm.at[pl.ds(base0 + j * CH, CH)], rows[b], rd[b])

    def rwait(b):
        pltpu.make_async_copy(eo_hbm.at[pl.ds(0, CH)], rows[b], rd[b]).wait()

    def sstart(j, b):
        pltpu.async_copy(rows[b], acc.at[ridx.at[j]], sa[b], add=True)

    def sdrain(b):
        pltpu.make_async_copy(rows[b], acc.at[ridx.at[0]], sa[b]).wait()

    for b in range(SSLOT):
        rstart(b, b)

    def body(i, carry):
        for b in range(SSLOT):
            @pl.when(lax.rem(i, SSLOT) == b)
            def _():
                rwait(b)
                sstart(i, b)

                @pl.when(i + SSLOT < NCHUNK)
                def _():
                    sdrain(b)
                    rstart(i + SSLOT, b)

        return carry

    lax.fori_loop(0, NCHUNK, body, 0)
    for b in range(SSLOT):
        sdrain(b)
    plsc.subcore_barrier()
    _writeback(seg_hbm)


@functools.partial(
    pl.kernel,
    out_type=jax.ShapeDtypeStruct((NC * N, DN), jnp.float32),
    mesh=_mesh,
    scratch_types=(
        [pltpu.VMEM((NCHUNK, CH), jnp.int32)]
        + [pltpu.VMEM((CH, DN), jnp.float32)]
        + [pltpu.SemaphoreType.DMA] * SSLOT
        + [pltpu.VMEM_SHARED((N, DN), jnp.float32)]
    ),
)
def _counts(rcv_hbm, zrow_hbm, ones_hbm, cnt_hbm, *refs):
    ridx = refs[0]
    buf = refs[1]
    sa = refs[2:2 + SSLOT]
    acc = refs[-1]

    cid = lax.axis_index("c")
    sid = lax.axis_index("s")
    wid = sid * NC + cid
    r0 = sid * RPT

    pltpu.sync_copy(rcv_hbm.at[wid], ridx)
    pltpu.sync_copy(zrow_hbm, buf)
    _ranged(sid, lambda k: pltpu.sync_copy(buf, acc.at[pl.ds(r0 + k * CH, CH)]))
    plsc.subcore_barrier()
    pltpu.sync_copy(ones_hbm, buf)

    def cstart(j, b):
        pltpu.async_copy(buf, acc.at[ridx.at[j]], sa[b], add=True)

    def cdrain(b):
        pltpu.make_async_copy(buf, acc.at[ridx.at[0]], sa[b]).wait()

    for b in range(SSLOT):
        cstart(b, b)

    def body(i, carry):
        for b in range(SSLOT):
            @pl.when(lax.rem(i, SSLOT) == b)
            def _():
                cdrain(b)

                @pl.when(i + SSLOT < NCHUNK)
                def _():
                    cstart(i + SSLOT, b)

        return carry

    lax.fori_loop(0, NCHUNK, body, 0)
    plsc.subcore_barrier()

    def wb(k):
        src = r0 + k * CH
        pltpu.sync_copy(acc.at[pl.ds(src, CH)], buf)
        pltpu.sync_copy(buf, cnt_hbm.at[pl.ds(cid * N + src, CH)])

    _ranged(sid, wb)


def _node_glob_body(nodes_ref, seg_ref, cnt_ref, g_ref,
                    nw0a_ref, nw0b_ref, nw0c_ref, nb0_ref,
                    nw1_ref, nb1_ref, nw2_ref, nb2_ref,
                    gw0a_ref, gw0b_ref, gw0c_ref, gb0_ref,
                    gw1_ref, gb1_ref, gw2_ref, gb2_ref, esum_ref,
                    nout_ref, gout_ref):
    seg = seg_ref[0] + seg_ref[1]
    cnt = cnt_ref[0, :, 0:1] + cnt_ref[1, :, 0:1]
    agg = seg / jnp.maximum(cnt, 1.0)
    g = g_ref[...]
    gterm = jnp.dot(g, nw0c_ref[...], preferred_element_type=jnp.float32) + nb0_ref[...]
    h0 = jnp.maximum(
        jnp.dot(nodes_ref[...], nw0a_ref[...], preferred_element_type=jnp.float32)
        + jnp.dot(agg, nw0b_ref[...], preferred_element_type=jnp.float32)
        + gterm, 0.0)
    h1 = jnp.maximum(
        jnp.dot(h0, nw1_ref[...], preferred_element_type=jnp.float32)
        + nb1_ref[...], 0.0)
    nout = jnp.dot(h1, nw2_ref[...], preferred_element_type=jnp.float32) + nb2_ref[...]
    nout_ref[...] = nout

    nmean = jnp.sum(nout, axis=0, keepdims=True) * (1.0 / N)
    emean = esum_ref[...] * (1.0 / E)
    x = jnp.maximum(
        jnp.dot(g, gw0a_ref[...], preferred_element_type=jnp.float32)
        + jnp.dot(nmean, gw0b_ref[...], preferred_element_type=jnp.float32)
        + jnp.dot(emean, gw0c_ref[...], preferred_element_type=jnp.float32)
        + gb0_ref[...], 0.0)
    x = jnp.maximum(
        jnp.dot(x, gw1_ref[...], preferred_element_type=jnp.float32)
        + gb1_ref[...], 0.0)
    gout_ref[...] = jnp.dot(x, gw2_ref[...], preferred_element_type=jnp.float32) + gb2_ref[...]


def _node_glob(nodes, seg, cnt, g, nw0a, nw0b, nw0c, nb0, nw1, nb1, nw2, nb2,
               gw0a, gw0b, gw0c, gb0, gw1, gb1, gw2, gb2, esum):
    return pl.pallas_call(
        _node_glob_body,
        out_shape=(
            jax.ShapeDtypeStruct((N, DN), jnp.float32),
            jax.ShapeDtypeStruct((1, DN), jnp.float32),
        ),
    )(nodes, seg, cnt, g, nw0a, nw0b, nw0c, nb0, nw1, nb1, nw2, nb2,
      gw0a, gw0b, gw0c, gb0, gw1, gb1, gw2, gb2, esum)


def kernel(nodes, edges, global_attr, senders, receivers,
           eW0, eb0, eW1, eb1, eW2, eb2,
           nW0, nb0, nW1, nb1, nW2, nb2,
           gW0, gb0, gW1, gb1, gW2, gb2):
    senders = senders.astype(jnp.int32)
    receivers = receivers.astype(jnp.int32)
    row = lambda b: b.reshape(1, -1)

    snd3 = senders.reshape(NW, NCHUNK, CH)
    rcv3 = receivers.reshape(NW, NCHUNK, CH)
    zrow = jnp.zeros((CH, DN), jnp.float32)
    ones = jnp.ones((CH, DN), jnp.float32)

    bf = jnp.bfloat16
    ps, pr = _proj(nodes, eW0[DE:DE + DN], eW0[DE + DN:DE + 2 * DN])
    gsum = _gather(ps, pr, snd3, rcv3)
    cnt = _counts(rcv3, zrow, ones)
    edges_out, esum = _edge_mlp(
        global_attr, eW0[DE + 2 * DN:], row(eb0), eW0[:DE].astype(bf),
        eW1.astype(bf), row(eb1), eW2.astype(bf), row(eb2),
        edges.astype(bf), gsum)

    seg = _scatter(edges_out, rcv3, zrow)
    seg = seg.reshape(NC, N, DN)
    cnt = cnt.reshape(NC, N, DN)

    nodes_out, global_out = _node_glob(
        nodes, seg, cnt, global_attr,
        nW0[:DN], nW0[DN:2 * DN], nW0[2 * DN:], row(nb0),
        nW1, row(nb1), nW2, row(nb2),
        gW0[:DN], gW0[DN:2 * DN], gW0[2 * DN:], row(gb0),
        gW1, row(gb1), gW2, row(gb2), esum)
    return (nodes_out, edges_out, global_out)
